# baseline (speedup 1.0000x reference)
import jax
import jax.numpy as jnp
from jax import lax
from jax.experimental import pallas as pl
from jax.experimental.pallas import tpu as pltpu

C_DIM = 192
QK = 192
HEADS = 8
CH = QK // HEADS
NWIN = 7
TOPK_N = 8
WH = 8
W2 = WH * WH
P2 = NWIN * NWIN
SIDE_N = 7
PAD = SIDE_N // 2
HW = NWIN * WH
N_PIX = HW * HW
KVC = QK + C_DIM
SCALE = QK ** -0.5
CHUNK = 112


def kernel(x, W_qkv, b_qkv, W_lepe, b_lepe, W_o, b_o):
    B, n, _ = x.shape
    x_sp = x.reshape(B, HW, HW, C_DIM)
    x_wm = (x_sp.reshape(B, NWIN, WH, NWIN, WH, C_DIM)
            .transpose(0, 1, 3, 2, 4, 5).reshape(B, N_PIX, C_DIM))
    wl = jnp.transpose(W_lepe[:, 0], (1, 2, 0))
    b_qkv2 = b_qkv.reshape(1, 3 * QK)
    bl2 = b_lepe.reshape(1, C_DIM)
    bo2 = b_o.reshape(1, C_DIM)

    def _proj_body(xwm_ref, w_ref, b_ref, qkv_ref, qm_ref, km_ref):
        for c in range(N_PIX // CHUNK):
            xc = xwm_ref[0, pl.ds(c * CHUNK, CHUNK), :]
            qkv_ref[0, pl.ds(c * CHUNK, CHUNK), :] = (
                jnp.dot(xc, w_ref[...], preferred_element_type=jnp.float32)
                + b_ref[0])
        for w in range(P2):
            blk = qkv_ref[0, w * W2:(w + 1) * W2, 0:2 * QK]
            qm_ref[0, w, 0] = jnp.sum(blk[:, :QK], axis=0) * (1.0 / W2)
            km_ref[0, w, 0] = jnp.sum(blk[:, QK:], axis=0) * (1.0 / W2)

    qkv_wm, qm, km = pl.pallas_call(
        _proj_body,
        grid=(B,),
        in_specs=[
            pl.BlockSpec((1, N_PIX, C_DIM), lambda b: (b, 0, 0)),
            pl.BlockSpec((C_DIM, 3 * QK), lambda b: (0, 0)),
            pl.BlockSpec((1, 3 * QK), lambda b: (0, 0)),
        ],
        out_specs=[
            pl.BlockSpec((1, N_PIX, 3 * QK), lambda b: (b, 0, 0)),
            pl.BlockSpec((1, P2, 1, QK), lambda b: (b, 0, 0, 0)),
            pl.BlockSpec((1, P2, 1, QK), lambda b: (b, 0, 0, 0)),
        ],
        out_shape=[
            jax.ShapeDtypeStruct((B, N_PIX, 3 * QK), jnp.float32),
            jax.ShapeDtypeStruct((B, P2, 1, QK), jnp.float32),
            jax.ShapeDtypeStruct((B, P2, 1, QK), jnp.float32),
        ],
        compiler_params=pltpu.CompilerParams(dimension_semantics=("parallel",)),
    )(x_wm, W_qkv, b_qkv2)

    def _route_body(qm_ref, km_ref, idx_ref):
        for b in range(B):
            qmb = qm_ref[b].reshape(P2, QK) * SCALE
            kmb = km_ref[b].reshape(P2, QK)
            logit = lax.dot_general(qmb, kmb, (((1,), (1,)), ((), ())),
                                    preferred_element_type=jnp.float32)
            iota = lax.broadcasted_iota(jnp.int32, (P2, P2), 1)
            l = logit
            cols = []
            for _ in range(TOPK_N):
                m = jnp.max(l, axis=1, keepdims=True)
                cand = jnp.where(l >= m, iota, 2 * P2)
                sel = jnp.min(cand, axis=1, keepdims=True)
                cols.append(sel)
                l = jnp.where(iota == sel, -1e30, l)
            idx_ref[b] = jnp.concatenate(cols, axis=1)

    r_idx = pl.pallas_call(
        _route_body,
        out_shape=jax.ShapeDtypeStruct((B, P2, TOPK_N), jnp.int32),
    )(qm, km)

    def _mega_body(idx_ref, qkv_ref, wl_ref, bl_ref, wo_ref, bo_ref, out_ref,
                   vpad_scr, lepe_scr, kv_scr):
        b = pl.program_id(0)

        vpad_scr[...] = jnp.zeros((HW + 2 * PAD, HW + 2 * PAD, C_DIM), jnp.float32)
        for w in range(P2):
            wj, wi = divmod(w, NWIN)
            vwin = qkv_ref[0, w * W2:(w + 1) * W2, 2 * QK:]
            vpad_scr[PAD + wj * WH:PAD + (wj + 1) * WH,
                     PAD + wi * WH:PAD + (wi + 1) * WH, :] = (
                vwin.reshape(WH, WH, C_DIM))

        for sj in range(NWIN):
            acc = jnp.zeros((WH, HW, C_DIM), jnp.float32)
            for dy in range(SIDE_N):
                row = vpad_scr[sj * WH + dy:sj * WH + dy + WH, :, :]
                for dx in range(SIDE_N):
                    acc += row[:, dx:dx + HW, :] * wl_ref[dy, dx]
            acc = acc + bl_ref[0]
            for wi in range(NWIN):
                lepe_scr[(sj * NWIN + wi) * W2:(sj * NWIN + wi + 1) * W2, :] = (
                    acc[:, wi * WH:(wi + 1) * WH, :].reshape(W2, C_DIM))

        def wbody(w, carry):
            q = qkv_ref[0, pl.ds(w * W2, W2), 0:QK].astype(jnp.bfloat16)
            for t in range(TOPK_N):
                r = idx_ref[b, w, t]
                kv_scr[t * W2:(t + 1) * W2, :] = (
                    qkv_ref[0, pl.ds(r * W2, W2), QK:].astype(jnp.bfloat16))
            outs = []
            for hh in range(HEADS):
                sl = slice(hh * CH, (hh + 1) * CH)
                slv = slice(QK + hh * CH, QK + (hh + 1) * CH)
                s = lax.dot_general(q[:, sl], kv_scr[:, sl], (((1,), (1,)), ((), ())),
                                    preferred_element_type=jnp.float32) * SCALE
                m = jnp.max(s, axis=1, keepdims=True)
                e = jnp.exp(s - m)
                p = (e / jnp.sum(e, axis=1, keepdims=True)).astype(jnp.bfloat16)
                outs.append(jnp.dot(p, kv_scr[:, slv], preferred_element_type=jnp.float32))
            o = jnp.concatenate(outs, axis=1)
            lep = lepe_scr[pl.ds(w * W2, W2), :]
            res = (jnp.dot(o + lep, wo_ref[...], preferred_element_type=jnp.float32)
                   + bo_ref[0])
            out_ref[0, pl.ds(w * W2, W2), :] = res
            return carry

        lax.fori_loop(0, P2, wbody, 0)

    grid_spec = pltpu.PrefetchScalarGridSpec(
        num_scalar_prefetch=1,
        grid=(B,),
        in_specs=[
            pl.BlockSpec((1, N_PIX, 3 * QK), lambda b, idx: (b, 0, 0)),
            pl.BlockSpec((SIDE_N, SIDE_N, C_DIM), lambda b, idx: (0, 0, 0)),
            pl.BlockSpec((1, C_DIM), lambda b, idx: (0, 0)),
            pl.BlockSpec((C_DIM, C_DIM), lambda b, idx: (0, 0)),
            pl.BlockSpec((1, C_DIM), lambda b, idx: (0, 0)),
        ],
        out_specs=pl.BlockSpec((1, N_PIX, C_DIM), lambda b, idx: (b, 0, 0)),
        scratch_shapes=[
            pltpu.VMEM((HW + 2 * PAD, HW + 2 * PAD, C_DIM), jnp.float32),
            pltpu.VMEM((N_PIX, C_DIM), jnp.float32),
            pltpu.VMEM((TOPK_N * W2, KVC), jnp.bfloat16),
        ],
    )
    out_wm = pl.pallas_call(
        _mega_body,
        grid_spec=grid_spec,
        out_shape=jax.ShapeDtypeStruct((B, N_PIX, C_DIM), jnp.float32),
        compiler_params=pltpu.CompilerParams(dimension_semantics=("arbitrary",)),
    )(r_idx, qkv_wm, wl, bl2, W_o, bo2)

    out_sp = (out_wm.reshape(B, NWIN, NWIN, WH, WH, C_DIM)
              .transpose(0, 1, 3, 2, 4, 5).reshape(B, n, C_DIM))
    return out_sp


# speedup vs baseline: 1.0458x; 1.0458x over previous
import jax
import jax.numpy as jnp
from jax import lax
from jax.experimental import pallas as pl
from jax.experimental.pallas import tpu as pltpu

C_DIM = 192
QK = 192
HEADS = 8
CH = QK // HEADS
NWIN = 7
TOPK_N = 8
WH = 8
W2 = WH * WH
P2 = NWIN * NWIN
SIDE_N = 7
PAD = SIDE_N // 2
HW = NWIN * WH
N_PIX = HW * HW
KVC = QK + C_DIM
SCALE = QK ** -0.5
CHUNK = 112


def kernel(x, W_qkv, b_qkv, W_lepe, b_lepe, W_o, b_o):
    B, n, _ = x.shape
    x_sp = x.reshape(B, HW, HW, C_DIM)
    x_wm = (x_sp.reshape(B, NWIN, WH, NWIN, WH, C_DIM)
            .transpose(0, 1, 3, 2, 4, 5).reshape(B, N_PIX, C_DIM))
    wl = jnp.transpose(W_lepe[:, 0], (1, 2, 0))
    b_qkv2 = b_qkv.reshape(1, 3 * QK)
    bl2 = b_lepe.reshape(1, C_DIM)
    bo2 = b_o.reshape(1, C_DIM)

    def _proj_body(xwm_ref, w_ref, b_ref, qkv_ref, qm_ref, km_ref):
        for c in range(N_PIX // CHUNK):
            xc = xwm_ref[0, pl.ds(c * CHUNK, CHUNK), :]
            qkv_ref[0, pl.ds(c * CHUNK, CHUNK), :] = (
                jnp.dot(xc, w_ref[...], preferred_element_type=jnp.float32)
                + b_ref[0])
        for w in range(P2):
            blk = qkv_ref[0, w * W2:(w + 1) * W2, 0:2 * QK]
            qm_ref[0, w, 0] = jnp.sum(blk[:, :QK], axis=0) * (1.0 / W2)
            km_ref[0, w, 0] = jnp.sum(blk[:, QK:], axis=0) * (1.0 / W2)

    qkv_wm, qm, km = pl.pallas_call(
        _proj_body,
        grid=(B,),
        in_specs=[
            pl.BlockSpec((1, N_PIX, C_DIM), lambda b: (b, 0, 0)),
            pl.BlockSpec((C_DIM, 3 * QK), lambda b: (0, 0)),
            pl.BlockSpec((1, 3 * QK), lambda b: (0, 0)),
        ],
        out_specs=[
            pl.BlockSpec((1, N_PIX, 3 * QK), lambda b: (b, 0, 0)),
            pl.BlockSpec((1, P2, 1, QK), lambda b: (b, 0, 0, 0)),
            pl.BlockSpec((1, P2, 1, QK), lambda b: (b, 0, 0, 0)),
        ],
        out_shape=[
            jax.ShapeDtypeStruct((B, N_PIX, 3 * QK), jnp.float32),
            jax.ShapeDtypeStruct((B, P2, 1, QK), jnp.float32),
            jax.ShapeDtypeStruct((B, P2, 1, QK), jnp.float32),
        ],
        compiler_params=pltpu.CompilerParams(dimension_semantics=("parallel",)),
    )(x_wm, W_qkv, b_qkv2)

    def _route_body(qm_ref, km_ref, idx_ref):
        for b in range(B):
            qmb = qm_ref[b].reshape(P2, QK) * SCALE
            kmb = km_ref[b].reshape(P2, QK)
            logit = lax.dot_general(qmb, kmb, (((1,), (1,)), ((), ())),
                                    preferred_element_type=jnp.float32)
            iota = lax.broadcasted_iota(jnp.int32, (P2, P2), 1)
            l = logit
            cols = []
            for _ in range(TOPK_N):
                m = jnp.max(l, axis=1, keepdims=True)
                cand = jnp.where(l >= m, iota, 2 * P2)
                sel = jnp.min(cand, axis=1, keepdims=True)
                cols.append(sel)
                l = jnp.where(iota == sel, -1e30, l)
            idx_ref[b] = jnp.concatenate(cols, axis=1)

    r_idx = pl.pallas_call(
        _route_body,
        out_shape=jax.ShapeDtypeStruct((B, P2, TOPK_N), jnp.int32),
    )(qm, km)

    def _mega_body(idx_ref, qkv_ref, wl_ref, bl_ref, wo_ref, bo_ref, out_ref,
                   vpad_scr, lepe_scr, kv_scr):
        b = pl.program_id(0)

        vpad_scr[...] = jnp.zeros((HW + 2 * PAD, HW + 2 * PAD, C_DIM), jnp.float32)
        for w in range(P2):
            wj, wi = divmod(w, NWIN)
            vwin = qkv_ref[0, w * W2:(w + 1) * W2, 2 * QK:]
            vpad_scr[PAD + wj * WH:PAD + (wj + 1) * WH,
                     PAD + wi * WH:PAD + (wi + 1) * WH, :] = (
                vwin.reshape(WH, WH, C_DIM))

        for sj in range(NWIN):
            acc = jnp.zeros((WH, HW, C_DIM), jnp.float32)
            for dy in range(SIDE_N):
                row = vpad_scr[sj * WH + dy:sj * WH + dy + WH, :, :]
                for dx in range(SIDE_N):
                    acc += row[:, dx:dx + HW, :] * wl_ref[dy, dx]
            acc = acc + bl_ref[0]
            for wi in range(NWIN):
                lepe_scr[(sj * NWIN + wi) * W2:(sj * NWIN + wi + 1) * W2, :] = (
                    acc[:, wi * WH:(wi + 1) * WH, :].reshape(W2, C_DIM))

        def wbody(w, carry):
            q = qkv_ref[0, pl.ds(w * W2, W2), 0:QK]
            for t in range(TOPK_N):
                r = idx_ref[b, w, t]
                kv_scr[t * W2:(t + 1) * W2, :] = qkv_ref[0, pl.ds(r * W2, W2), QK:]
            outs = []
            for hh in range(HEADS):
                sl = slice(hh * CH, (hh + 1) * CH)
                slv = slice(QK + hh * CH, QK + (hh + 1) * CH)
                s = lax.dot_general(q[:, sl], kv_scr[:, sl], (((1,), (1,)), ((), ())),
                                    preferred_element_type=jnp.float32) * SCALE
                m = jnp.max(s, axis=1, keepdims=True)
                e = jnp.exp(s - m)
                p = e / jnp.sum(e, axis=1, keepdims=True)
                outs.append(jnp.dot(p, kv_scr[:, slv], preferred_element_type=jnp.float32))
            o = jnp.concatenate(outs, axis=1)
            lep = lepe_scr[pl.ds(w * W2, W2), :]
            res = (jnp.dot(o + lep, wo_ref[...], preferred_element_type=jnp.float32)
                   + bo_ref[0])
            out_ref[0, pl.ds(w * W2, W2), :] = res
            return carry

        lax.fori_loop(0, P2, wbody, 0)

    grid_spec = pltpu.PrefetchScalarGridSpec(
        num_scalar_prefetch=1,
        grid=(B,),
        in_specs=[
            pl.BlockSpec((1, N_PIX, 3 * QK), lambda b, idx: (b, 0, 0)),
            pl.BlockSpec((SIDE_N, SIDE_N, C_DIM), lambda b, idx: (0, 0, 0)),
            pl.BlockSpec((1, C_DIM), lambda b, idx: (0, 0)),
            pl.BlockSpec((C_DIM, C_DIM), lambda b, idx: (0, 0)),
            pl.BlockSpec((1, C_DIM), lambda b, idx: (0, 0)),
        ],
        out_specs=pl.BlockSpec((1, N_PIX, C_DIM), lambda b, idx: (b, 0, 0)),
        scratch_shapes=[
            pltpu.VMEM((HW + 2 * PAD, HW + 2 * PAD, C_DIM), jnp.float32),
            pltpu.VMEM((N_PIX, C_DIM), jnp.float32),
            pltpu.VMEM((TOPK_N * W2, KVC), jnp.float32),
        ],
    )
    out_wm = pl.pallas_call(
        _mega_body,
        grid_spec=grid_spec,
        out_shape=jax.ShapeDtypeStruct((B, N_PIX, C_DIM), jnp.float32),
        compiler_params=pltpu.CompilerParams(dimension_semantics=("parallel",)),
    )(r_idx, qkv_wm, wl, bl2, W_o, bo2)

    out_sp = (out_wm.reshape(B, NWIN, NWIN, WH, WH, C_DIM)
              .transpose(0, 1, 3, 2, 4, 5).reshape(B, n, C_DIM))
    return out_sp
